# Initial kernel scaffold; baseline (speedup 1.0000x reference)
import jax
import jax.numpy as jnp
from jax.experimental import pallas as pl


def kernel(x, edge_index, batch, W1, b1, g1, be1, W2, b2, g2, be2, W3, b3, g3, be3, fc1_W, fc1_b, fc2_W, fc2_b):
    raise NotImplementedError("write your pallas kernel here")


# speedup vs baseline: 10.9170x; 10.9170x over previous
import functools

import jax
import jax.numpy as jnp
from jax import lax
from jax.experimental import pallas as pl
from jax.experimental.pallas import tpu as pltpu
from jax.experimental.pallas import tpu_sc as plsc

N = 50000
NP = 50176
E = 800000
R = 6400
H = 64
NG = 32
EPS = 1e-5
BN = 6272
NBLK = NP // BN
NPT = NP // 16
K = 8

_f32 = jnp.float32
_i32 = jnp.int32


def _sc_mesh():
    return plsc.VectorSubcoreMesh(core_axis_name="c", subcore_axis_name="s")


def _sc_deg(dst2):
    @functools.partial(
        pl.kernel,
        out_type=jax.ShapeDtypeStruct((2 * NP,), _f32),
        mesh=_sc_mesh(),
        compiler_params=pltpu.CompilerParams(use_tc_tiling_on_sc=False),
        scratch_types=[
            pltpu.VMEM((NPT,), _f32),
            pltpu.VMEM((128,), _f32),
            pltpu.VMEM((K, 128), _i32),
            pltpu.VMEM_SHARED((NP,), _f32),
        ],
    )
    def k(dst2_hbm, degp_hbm, zbuf, ones_v, dstv, degp_sh):
        c = lax.axis_index("c")
        s = lax.axis_index("s")
        @pl.loop(0, NPT // 16)
        def _z(i):
            zbuf[pl.ds(i * 16, 16)] = jnp.zeros((16,), _f32)
        for j in range(8):
            ones_v[pl.ds(j * 16, 16)] = jnp.ones((16,), _f32)
        base_n = s * NPT
        pltpu.sync_copy(zbuf, degp_sh.at[pl.ds(base_n, NPT)])
        plsc.subcore_barrier()
        rows_per_tile = R // 32
        base_r = c * (R // 2) + s * rows_per_tile
        @pl.loop(0, rows_per_tile // K)
        def _chunk(kk):
            r0 = base_r + kk * K
            pltpu.sync_copy(dst2_hbm.at[pl.ds(r0, K)], dstv)
            for j in range(K):
                pltpu.sync_copy(ones_v.at[pl.ds(0, 128)],
                                degp_sh.at[dstv.at[j]], add=True)
        plsc.subcore_barrier()
        pltpu.sync_copy(degp_sh.at[pl.ds(base_n, NPT)], zbuf)
        pltpu.sync_copy(zbuf, degp_hbm.at[pl.ds(c * NP + base_n, NPT)])

    return k(dst2).reshape(2, NP)


def _sc_agg(ts_flat, srcoff4, dst2):
    @functools.partial(
        pl.kernel,
        out_type=jax.ShapeDtypeStruct((4 * NP, 16), _f32),
        mesh=_sc_mesh(),
        compiler_params=pltpu.CompilerParams(use_tc_tiling_on_sc=False),
        scratch_types=[
            pltpu.VMEM((K * 128, 16), _f32),
            pltpu.VMEM((K, 128), _i32),
            pltpu.VMEM((K, 128), _i32),
            pltpu.SemaphoreType.DMA,
            pltpu.VMEM_SHARED((NP, 16), _f32),
        ],
    )
    def k(ts_hbm, src_hbm, dst_hbm, out_hbm, rows_v, srcv, dstv, sem, acc):
        c = lax.axis_index("c")
        s = lax.axis_index("s")
        base_n = s * NPT
        ch = NPT // 4
        rows_per_tile = R // 16
        base_r = s * rows_per_tile
        for p in range(2):
            g = 2 * c + p
            for q in range(4):
                pltpu.sync_copy(
                    ts_hbm.at[pl.ds(g * NP + base_n + q * ch, ch)],
                    rows_v.at[pl.ds(0, ch)])
                pltpu.sync_copy(rows_v.at[pl.ds(0, ch)],
                                acc.at[pl.ds(base_n + q * ch, ch)])
            plsc.subcore_barrier()
            @pl.loop(0, rows_per_tile // K)
            def _chunk(kk):
                r0 = base_r + kk * K
                pltpu.sync_copy(src_hbm.at[g, pl.ds(r0, K)], srcv)
                pltpu.sync_copy(dst_hbm.at[pl.ds(r0, K)], dstv)
                cps = [pltpu.async_copy(ts_hbm.at[srcv.at[j]],
                                        rows_v.at[pl.ds(j * 128, 128)], sem)
                       for j in range(K)]
                for cp in cps:
                    cp.wait()
                for j in range(K):
                    pltpu.sync_copy(rows_v.at[pl.ds(j * 128, 128)],
                                    acc.at[dstv.at[j]], add=True)
            plsc.subcore_barrier()
            for q in range(4):
                pltpu.sync_copy(acc.at[pl.ds(base_n + q * ch, ch)],
                                rows_v.at[pl.ds(0, ch)])
                pltpu.sync_copy(
                    rows_v.at[pl.ds(0, ch)],
                    out_hbm.at[pl.ds(g * NP + base_n + q * ch, ch)])

    return k(ts_flat, srcoff4, dst2)


def _split4(ts):
    return jnp.concatenate(
        [ts[None, :, 16 * g:16 * (g + 1)] for g in range(4)], 0)


def _tc_prep(degpT, xp, W1p):
    def body(degp_ref, x_ref, w_ref, dinv_ref, ts_ref):
        deg = degp_ref[:, 0:1] + degp_ref[:, 1:2] + 1.0
        di = 1.0 / jnp.sqrt(deg)
        dinv_ref[...] = di
        t = jnp.dot(x_ref[...], w_ref[...], preferred_element_type=_f32, precision=lax.Precision.HIGHEST)
        ts_ref[...] = _split4(t * di)

    return pl.pallas_call(
        body,
        grid=(NBLK,),
        in_specs=[
            pl.BlockSpec((BN, 2), lambda i: (i, 0)),
            pl.BlockSpec((BN, 8), lambda i: (i, 0)),
            pl.BlockSpec((8, 64), lambda i: (0, 0)),
        ],
        out_specs=[
            pl.BlockSpec((BN, 1), lambda i: (i, 0)),
            pl.BlockSpec((4, BN, 16), lambda i: (0, i, 0)),
        ],
        out_shape=[
            jax.ShapeDtypeStruct((NP, 1), _f32),
            jax.ShapeDtypeStruct((4, NP, 16), _f32),
        ],
    )(degpT, xp, W1p)


def _tc_stats(agg, dinv, b):
    def body(a0, a1, a2, a3, dinv_ref, b_ref, pre_ref, st_ref):
        i = pl.program_id(0)
        a = jnp.concatenate([a0[0], a1[0], a2[0], a3[0]], axis=-1)
        pre = a * dinv_ref[...] + b_ref[...]
        pre_ref[...] = pre
        rows = i * BN + lax.broadcasted_iota(_i32, (BN, 1), 0)
        pm = jnp.where(rows < N, pre, 0.0)
        st = jnp.concatenate([jnp.sum(pm, 0)[None], jnp.sum(pm * pm, 0)[None]],
                             0)
        @pl.when(i == 0)
        def _():
            st_ref[...] = st
        @pl.when(i > 0)
        def _():
            st_ref[...] += st

    gspec = lambda g: pl.BlockSpec((1, BN, 16), lambda i, g=g: (g, i, 0))
    return pl.pallas_call(
        body,
        grid=(NBLK,),
        in_specs=[
            gspec(0), gspec(1), gspec(2), gspec(3),
            pl.BlockSpec((BN, 1), lambda i: (i, 0)),
            pl.BlockSpec((1, 64), lambda i: (0, 0)),
        ],
        out_specs=[
            pl.BlockSpec((BN, 64), lambda i: (i, 0)),
            pl.BlockSpec((2, 64), lambda i: (0, 0)),
        ],
        out_shape=[
            jax.ShapeDtypeStruct((NP, 64), _f32),
            jax.ShapeDtypeStruct((2, 64), _f32),
        ],
    )(agg, agg, agg, agg, dinv, b)


def _bn_relu(pre_ref, st_ref, g_ref, be_ref, i):
    m = st_ref[0:1, :] * (1.0 / N)
    v = st_ref[1:2, :] * (1.0 / N) - m * m
    scale = g_ref[...] / jnp.sqrt(v + EPS)
    shift = be_ref[...] - m * scale
    h = jnp.maximum(pre_ref[...] * scale + shift, 0.0)
    rows = i * BN + lax.broadcasted_iota(_i32, (BN, 1), 0)
    return jnp.where(rows < N, h, 0.0)


def _tc_apply_mid(pre, stats, dinv, g, be, Wn):
    def body(pre_ref, st_ref, dinv_ref, g_ref, be_ref, w_ref, ts_ref):
        i = pl.program_id(0)
        h = _bn_relu(pre_ref, st_ref, g_ref, be_ref, i)
        t = jnp.dot(h, w_ref[...], preferred_element_type=_f32, precision=lax.Precision.HIGHEST)
        ts_ref[...] = _split4(t * dinv_ref[...])

    return pl.pallas_call(
        body,
        grid=(NBLK,),
        in_specs=[
            pl.BlockSpec((BN, 64), lambda i: (i, 0)),
            pl.BlockSpec((2, 64), lambda i: (0, 0)),
            pl.BlockSpec((BN, 1), lambda i: (i, 0)),
            pl.BlockSpec((1, 64), lambda i: (0, 0)),
            pl.BlockSpec((1, 64), lambda i: (0, 0)),
            pl.BlockSpec((64, 64), lambda i: (0, 0)),
        ],
        out_specs=[pl.BlockSpec((4, BN, 16), lambda i: (0, i, 0))],
        out_shape=[jax.ShapeDtypeStruct((4, NP, 16), _f32)],
    )(pre, stats, dinv, g, be, Wn)[0]


def _tc_final(pre, stats, g, be, batch2, fc1_W, fc1_b, fc2p, fc2bp):
    def body(pre_ref, st_ref, g_ref, be_ref, bt_ref, w1_ref, b1_ref, w2_ref,
             b2_ref, pool_ref, cnt_ref, out_ref):
        i = pl.program_id(0)
        h = _bn_relu(pre_ref, st_ref, g_ref, be_ref, i)
        oh = (bt_ref[...] == lax.broadcasted_iota(_i32, (1, NG), 1))
        oh = oh.astype(_f32)
        psum = lax.dot_general(oh, h, (((0,), (0,)), ((), ())),
                               preferred_element_type=_f32, precision=lax.Precision.HIGHEST)
        csum = lax.dot_general(oh, jnp.ones((BN, H), _f32),
                               (((0,), (0,)), ((), ())),
                               preferred_element_type=_f32, precision=lax.Precision.HIGHEST)
        @pl.when(i == 0)
        def _():
            pool_ref[...] = psum
            cnt_ref[...] = csum
        @pl.when(i > 0)
        def _():
            pool_ref[...] += psum
            cnt_ref[...] += csum
        pooled = pool_ref[...] / jnp.maximum(cnt_ref[...], 1.0)
        z = jnp.maximum(
            jnp.dot(pooled, w1_ref[...], preferred_element_type=_f32, precision=lax.Precision.HIGHEST)
            + b1_ref[...], 0.0)
        out_ref[...] = jnp.dot(z, w2_ref[...],
                               preferred_element_type=_f32, precision=lax.Precision.HIGHEST) + b2_ref[...]

    return pl.pallas_call(
        body,
        grid=(NBLK,),
        in_specs=[
            pl.BlockSpec((BN, 64), lambda i: (i, 0)),
            pl.BlockSpec((2, 64), lambda i: (0, 0)),
            pl.BlockSpec((1, 64), lambda i: (0, 0)),
            pl.BlockSpec((1, 64), lambda i: (0, 0)),
            pl.BlockSpec((BN, 1), lambda i: (i, 0)),
            pl.BlockSpec((64, 64), lambda i: (0, 0)),
            pl.BlockSpec((1, 64), lambda i: (0, 0)),
            pl.BlockSpec((64, 128), lambda i: (0, 0)),
            pl.BlockSpec((1, 128), lambda i: (0, 0)),
        ],
        out_specs=[
            pl.BlockSpec((NG, 64), lambda i: (0, 0)),
            pl.BlockSpec((NG, 64), lambda i: (0, 0)),
            pl.BlockSpec((NG, 128), lambda i: (0, 0)),
        ],
        out_shape=[
            jax.ShapeDtypeStruct((NG, 64), _f32),
            jax.ShapeDtypeStruct((NG, 64), _f32),
            jax.ShapeDtypeStruct((NG, 128), _f32),
        ],
    )(pre, stats, g, be, batch2, fc1_W, fc1_b, fc2p, fc2bp)[2]


def kernel(x, edge_index, batch, W1, b1, g1, be1, W2, b2, g2, be2, W3, b3, g3,
           be3, fc1_W, fc1_b, fc2_W, fc2_b):
    src = edge_index[0]
    dst = edge_index[1]
    epad = R * 128 - E
    src2 = jnp.concatenate([src, jnp.full((epad,), N, _i32)]).reshape(R, 128)
    dst2 = jnp.concatenate([dst, jnp.full((epad,), N, _i32)]).reshape(R, 128)
    srcoff4 = jnp.concatenate([(src2 + g * NP)[None] for g in range(4)], 0)

    xp = jnp.pad(x, ((0, NP - N), (0, 2)))
    W1p = jnp.pad(W1, ((0, 2), (0, 0)))
    batch2 = jnp.pad(batch, (0, NP - N), constant_values=NG).reshape(NP, 1)
    b1r, b2r, b3r = b1.reshape(1, H), b2.reshape(1, H), b3.reshape(1, H)
    g1r, g2r, g3r = g1.reshape(1, H), g2.reshape(1, H), g3.reshape(1, H)
    be1r, be2r, be3r = be1.reshape(1, H), be2.reshape(1, H), be3.reshape(1, H)
    fc1b = fc1_b.reshape(1, H)
    fc2p = jnp.pad(fc2_W, ((0, 0), (0, 127)))
    fc2bp = jnp.pad(fc2_b.reshape(1, 1), ((0, 0), (0, 127)))

    degp = _sc_deg(dst2)
    dinv, ts = _tc_prep(degp.T, xp, W1p)

    agg1 = _sc_agg(ts.reshape(4 * NP, 16), srcoff4, dst2).reshape(4, NP, 16)
    pre1, st1 = _tc_stats(agg1, dinv, b1r)
    ts2 = _tc_apply_mid(pre1, st1, dinv, g1r, be1r, W2)

    agg2 = _sc_agg(ts2.reshape(4 * NP, 16), srcoff4, dst2).reshape(4, NP, 16)
    pre2, st2 = _tc_stats(agg2, dinv, b2r)
    ts3 = _tc_apply_mid(pre2, st2, dinv, g2r, be2r, W3)

    agg3 = _sc_agg(ts3.reshape(4 * NP, 16), srcoff4, dst2).reshape(4, NP, 16)
    pre3, st3 = _tc_stats(agg3, dinv, b3r)
    out = _tc_final(pre3, st3, g3r, be3r, batch2, fc1_W, fc1b, fc2p, fc2bp)
    return out[:, :1]


# speedup vs baseline: 14.1138x; 1.2928x over previous
import functools

import jax
import jax.numpy as jnp
from jax import lax
from jax.experimental import pallas as pl
from jax.experimental.pallas import tpu as pltpu
from jax.experimental.pallas import tpu_sc as plsc

N = 50000
NP = 50176
E = 800000
R = 6400
H = 64
NG = 32
EPS = 1e-5
BN = 6272
NBLK = NP // BN
NPT = NP // 16
K = 8

_f32 = jnp.float32
_i32 = jnp.int32


def _sc_mesh():
    return plsc.VectorSubcoreMesh(core_axis_name="c", subcore_axis_name="s")


def _sc_deg(dst2):
    @functools.partial(
        pl.kernel,
        out_type=jax.ShapeDtypeStruct((2 * NP,), _f32),
        mesh=_sc_mesh(),
        compiler_params=pltpu.CompilerParams(use_tc_tiling_on_sc=False),
        scratch_types=[
            pltpu.VMEM((NPT,), _f32),
            pltpu.VMEM((128,), _f32),
            pltpu.VMEM((K, 128), _i32),
            pltpu.VMEM_SHARED((NP,), _f32),
        ],
    )
    def k(dst2_hbm, degp_hbm, zbuf, ones_v, dstv, degp_sh):
        c = lax.axis_index("c")
        s = lax.axis_index("s")
        @pl.loop(0, NPT // 16)
        def _z(i):
            zbuf[pl.ds(i * 16, 16)] = jnp.zeros((16,), _f32)
        for j in range(8):
            ones_v[pl.ds(j * 16, 16)] = jnp.ones((16,), _f32)
        base_n = s * NPT
        pltpu.sync_copy(zbuf, degp_sh.at[pl.ds(base_n, NPT)])
        plsc.subcore_barrier()
        rows_per_tile = R // 32
        base_r = c * (R // 2) + s * rows_per_tile
        @pl.loop(0, rows_per_tile // K)
        def _chunk(kk):
            r0 = base_r + kk * K
            pltpu.sync_copy(dst2_hbm.at[pl.ds(r0, K)], dstv)
            for j in range(K):
                pltpu.sync_copy(ones_v.at[pl.ds(0, 128)],
                                degp_sh.at[dstv.at[j]], add=True)
        plsc.subcore_barrier()
        pltpu.sync_copy(degp_sh.at[pl.ds(base_n, NPT)], zbuf)
        pltpu.sync_copy(zbuf, degp_hbm.at[pl.ds(c * NP + base_n, NPT)])

    return k(dst2).reshape(2, NP)


def _sc_agg(ts_flat, srcoff4, dst2):
    @functools.partial(
        pl.kernel,
        out_type=jax.ShapeDtypeStruct((4 * NP, 16), _f32),
        mesh=_sc_mesh(),
        compiler_params=pltpu.CompilerParams(use_tc_tiling_on_sc=False),
        scratch_types=[
            pltpu.VMEM((2 * K * 128, 16), _f32),
            pltpu.VMEM((2 * K, 128), _i32),
            pltpu.VMEM((2 * K, 128), _i32),
            pltpu.SemaphoreType.DMA,
            pltpu.SemaphoreType.DMA,
            pltpu.SemaphoreType.DMA,
            pltpu.SemaphoreType.DMA,
            pltpu.VMEM_SHARED((NP, 16), _f32),
        ],
    )
    def k(ts_hbm, src_hbm, dst_hbm, out_hbm, rows_v, srcv, dstv,
          gsem0, gsem1, ssem0, ssem1, acc):
        c = lax.axis_index("c")
        s = lax.axis_index("s")
        gsem = (gsem0, gsem1)
        ssem = (ssem0, ssem1)
        base_n = s * NPT
        ch = NPT // 4
        rows_per_tile = R // 16
        nchunk = rows_per_tile // K
        base_r = s * rows_per_tile
        for p in range(2):
            g = 2 * c + p

            def load_idx(kk, b):
                r0 = base_r + kk * K
                pltpu.sync_copy(src_hbm.at[g, pl.ds(r0, K)],
                                srcv.at[pl.ds(b * K, K)])
                pltpu.sync_copy(dst_hbm.at[pl.ds(r0, K)],
                                dstv.at[pl.ds(b * K, K)])

            def gathers(b, fire):
                for j in range(K):
                    cp = pltpu.make_async_copy(
                        ts_hbm.at[srcv.at[b * K + j]],
                        rows_v.at[pl.ds((b * K + j) * 128, 128)], gsem[b])
                    cp.start() if fire else cp.wait()

            def scatters(b, fire):
                for j in range(K):
                    src_sl = rows_v.at[pl.ds((b * K + j) * 128, 128)]
                    dst_sl = acc.at[dstv.at[b * K + j]]
                    if fire:
                        pltpu.async_copy(src_sl, dst_sl, ssem[b], add=True)
                    else:
                        pltpu.make_async_copy(src_sl, dst_sl, ssem[b]).wait()

            def body(kk, b):
                nb = 1 - b
                scatters(nb, False)
                load_idx(kk + 1, nb)
                gathers(nb, True)
                gathers(b, False)
                scatters(b, True)

            for q in range(4):
                pltpu.sync_copy(
                    ts_hbm.at[pl.ds(g * NP + base_n + q * ch, ch)],
                    rows_v.at[pl.ds(0, ch)])
                pltpu.sync_copy(rows_v.at[pl.ds(0, ch)],
                                acc.at[pl.ds(base_n + q * ch, ch)])
            plsc.subcore_barrier()
            load_idx(0, 0)
            gathers(0, True)
            load_idx(1, 1)
            gathers(1, True)
            gathers(0, False)
            scatters(0, True)
            scatters(0, False)
            load_idx(2, 0)
            gathers(0, True)
            gathers(1, False)
            scatters(1, True)
            @pl.loop(0, (nchunk - 4) // 2)
            def _pair(it):
                kk = 2 + 2 * it
                body(kk, 0)
                body(kk + 1, 1)
            scatters(1, False)
            load_idx(nchunk - 1, 1)
            gathers(1, True)
            gathers(0, False)
            scatters(0, True)
            scatters(0, False)
            gathers(1, False)
            scatters(1, True)
            scatters(1, False)
            plsc.subcore_barrier()
            for q in range(4):
                pltpu.sync_copy(acc.at[pl.ds(base_n + q * ch, ch)],
                                rows_v.at[pl.ds(0, ch)])
                pltpu.sync_copy(
                    rows_v.at[pl.ds(0, ch)],
                    out_hbm.at[pl.ds(g * NP + base_n + q * ch, ch)])

    return k(ts_flat, srcoff4, dst2)


def _split4(ts):
    return jnp.concatenate(
        [ts[None, :, 16 * g:16 * (g + 1)] for g in range(4)], 0)


def _tc_prep(degpT, xp, W1p):
    def body(degp_ref, x_ref, w_ref, dinv_ref, ts_ref):
        deg = degp_ref[:, 0:1] + degp_ref[:, 1:2] + 1.0
        di = 1.0 / jnp.sqrt(deg)
        dinv_ref[...] = di
        t = jnp.dot(x_ref[...], w_ref[...], preferred_element_type=_f32, precision=lax.Precision.HIGHEST)
        ts_ref[...] = _split4(t * di)

    return pl.pallas_call(
        body,
        grid=(NBLK,),
        in_specs=[
            pl.BlockSpec((BN, 2), lambda i: (i, 0)),
            pl.BlockSpec((BN, 8), lambda i: (i, 0)),
            pl.BlockSpec((8, 64), lambda i: (0, 0)),
        ],
        out_specs=[
            pl.BlockSpec((BN, 1), lambda i: (i, 0)),
            pl.BlockSpec((4, BN, 16), lambda i: (0, i, 0)),
        ],
        out_shape=[
            jax.ShapeDtypeStruct((NP, 1), _f32),
            jax.ShapeDtypeStruct((4, NP, 16), _f32),
        ],
    )(degpT, xp, W1p)


def _tc_stats(agg, dinv, b):
    def body(a0, a1, a2, a3, dinv_ref, b_ref, pre_ref, st_ref):
        i = pl.program_id(0)
        a = jnp.concatenate([a0[0], a1[0], a2[0], a3[0]], axis=-1)
        pre = a * dinv_ref[...] + b_ref[...]
        pre_ref[...] = pre
        rows = i * BN + lax.broadcasted_iota(_i32, (BN, 1), 0)
        pm = jnp.where(rows < N, pre, 0.0)
        st = jnp.concatenate([jnp.sum(pm, 0)[None], jnp.sum(pm * pm, 0)[None]],
                             0)
        @pl.when(i == 0)
        def _():
            st_ref[...] = st
        @pl.when(i > 0)
        def _():
            st_ref[...] += st

    gspec = lambda g: pl.BlockSpec((1, BN, 16), lambda i, g=g: (g, i, 0))
    return pl.pallas_call(
        body,
        grid=(NBLK,),
        in_specs=[
            gspec(0), gspec(1), gspec(2), gspec(3),
            pl.BlockSpec((BN, 1), lambda i: (i, 0)),
            pl.BlockSpec((1, 64), lambda i: (0, 0)),
        ],
        out_specs=[
            pl.BlockSpec((BN, 64), lambda i: (i, 0)),
            pl.BlockSpec((2, 64), lambda i: (0, 0)),
        ],
        out_shape=[
            jax.ShapeDtypeStruct((NP, 64), _f32),
            jax.ShapeDtypeStruct((2, 64), _f32),
        ],
    )(agg, agg, agg, agg, dinv, b)


def _bn_relu(pre_ref, st_ref, g_ref, be_ref, i):
    m = st_ref[0:1, :] * (1.0 / N)
    v = st_ref[1:2, :] * (1.0 / N) - m * m
    scale = g_ref[...] / jnp.sqrt(v + EPS)
    shift = be_ref[...] - m * scale
    h = jnp.maximum(pre_ref[...] * scale + shift, 0.0)
    rows = i * BN + lax.broadcasted_iota(_i32, (BN, 1), 0)
    return jnp.where(rows < N, h, 0.0)


def _tc_apply_mid(pre, stats, dinv, g, be, Wn):
    def body(pre_ref, st_ref, dinv_ref, g_ref, be_ref, w_ref, ts_ref):
        i = pl.program_id(0)
        h = _bn_relu(pre_ref, st_ref, g_ref, be_ref, i)
        t = jnp.dot(h, w_ref[...], preferred_element_type=_f32, precision=lax.Precision.HIGHEST)
        ts_ref[...] = _split4(t * dinv_ref[...])

    return pl.pallas_call(
        body,
        grid=(NBLK,),
        in_specs=[
            pl.BlockSpec((BN, 64), lambda i: (i, 0)),
            pl.BlockSpec((2, 64), lambda i: (0, 0)),
            pl.BlockSpec((BN, 1), lambda i: (i, 0)),
            pl.BlockSpec((1, 64), lambda i: (0, 0)),
            pl.BlockSpec((1, 64), lambda i: (0, 0)),
            pl.BlockSpec((64, 64), lambda i: (0, 0)),
        ],
        out_specs=[pl.BlockSpec((4, BN, 16), lambda i: (0, i, 0))],
        out_shape=[jax.ShapeDtypeStruct((4, NP, 16), _f32)],
    )(pre, stats, dinv, g, be, Wn)[0]


def _tc_final(pre, stats, g, be, batch2, fc1_W, fc1_b, fc2p, fc2bp):
    def body(pre_ref, st_ref, g_ref, be_ref, bt_ref, w1_ref, b1_ref, w2_ref,
             b2_ref, pool_ref, cnt_ref, out_ref):
        i = pl.program_id(0)
        h = _bn_relu(pre_ref, st_ref, g_ref, be_ref, i)
        oh = (bt_ref[...] == lax.broadcasted_iota(_i32, (1, NG), 1))
        oh = oh.astype(_f32)
        psum = lax.dot_general(oh, h, (((0,), (0,)), ((), ())),
                               preferred_element_type=_f32, precision=lax.Precision.HIGHEST)
        csum = lax.dot_general(oh, jnp.ones((BN, H), _f32),
                               (((0,), (0,)), ((), ())),
                               preferred_element_type=_f32, precision=lax.Precision.HIGHEST)
        @pl.when(i == 0)
        def _():
            pool_ref[...] = psum
            cnt_ref[...] = csum
        @pl.when(i > 0)
        def _():
            pool_ref[...] += psum
            cnt_ref[...] += csum
        pooled = pool_ref[...] / jnp.maximum(cnt_ref[...], 1.0)
        z = jnp.maximum(
            jnp.dot(pooled, w1_ref[...], preferred_element_type=_f32, precision=lax.Precision.HIGHEST)
            + b1_ref[...], 0.0)
        out_ref[...] = jnp.dot(z, w2_ref[...],
                               preferred_element_type=_f32, precision=lax.Precision.HIGHEST) + b2_ref[...]

    return pl.pallas_call(
        body,
        grid=(NBLK,),
        in_specs=[
            pl.BlockSpec((BN, 64), lambda i: (i, 0)),
            pl.BlockSpec((2, 64), lambda i: (0, 0)),
            pl.BlockSpec((1, 64), lambda i: (0, 0)),
            pl.BlockSpec((1, 64), lambda i: (0, 0)),
            pl.BlockSpec((BN, 1), lambda i: (i, 0)),
            pl.BlockSpec((64, 64), lambda i: (0, 0)),
            pl.BlockSpec((1, 64), lambda i: (0, 0)),
            pl.BlockSpec((64, 128), lambda i: (0, 0)),
            pl.BlockSpec((1, 128), lambda i: (0, 0)),
        ],
        out_specs=[
            pl.BlockSpec((NG, 64), lambda i: (0, 0)),
            pl.BlockSpec((NG, 64), lambda i: (0, 0)),
            pl.BlockSpec((NG, 128), lambda i: (0, 0)),
        ],
        out_shape=[
            jax.ShapeDtypeStruct((NG, 64), _f32),
            jax.ShapeDtypeStruct((NG, 64), _f32),
            jax.ShapeDtypeStruct((NG, 128), _f32),
        ],
    )(pre, stats, g, be, batch2, fc1_W, fc1_b, fc2p, fc2bp)[2]


def kernel(x, edge_index, batch, W1, b1, g1, be1, W2, b2, g2, be2, W3, b3, g3,
           be3, fc1_W, fc1_b, fc2_W, fc2_b):
    src = edge_index[0]
    dst = edge_index[1]
    epad = R * 128 - E
    src2 = jnp.concatenate([src, jnp.full((epad,), N, _i32)]).reshape(R, 128)
    dst2 = jnp.concatenate([dst, jnp.full((epad,), N, _i32)]).reshape(R, 128)
    srcoff4 = jnp.concatenate([(src2 + g * NP)[None] for g in range(4)], 0)

    xp = jnp.pad(x, ((0, NP - N), (0, 2)))
    W1p = jnp.pad(W1, ((0, 2), (0, 0)))
    batch2 = jnp.pad(batch, (0, NP - N), constant_values=NG).reshape(NP, 1)
    b1r, b2r, b3r = b1.reshape(1, H), b2.reshape(1, H), b3.reshape(1, H)
    g1r, g2r, g3r = g1.reshape(1, H), g2.reshape(1, H), g3.reshape(1, H)
    be1r, be2r, be3r = be1.reshape(1, H), be2.reshape(1, H), be3.reshape(1, H)
    fc1b = fc1_b.reshape(1, H)
    fc2p = jnp.pad(fc2_W, ((0, 0), (0, 127)))
    fc2bp = jnp.pad(fc2_b.reshape(1, 1), ((0, 0), (0, 127)))

    degp = _sc_deg(dst2)
    dinv, ts = _tc_prep(degp.T, xp, W1p)

    agg1 = _sc_agg(ts.reshape(4 * NP, 16), srcoff4, dst2).reshape(4, NP, 16)
    pre1, st1 = _tc_stats(agg1, dinv, b1r)
    ts2 = _tc_apply_mid(pre1, st1, dinv, g1r, be1r, W2)

    agg2 = _sc_agg(ts2.reshape(4 * NP, 16), srcoff4, dst2).reshape(4, NP, 16)
    pre2, st2 = _tc_stats(agg2, dinv, b2r)
    ts3 = _tc_apply_mid(pre2, st2, dinv, g2r, be2r, W3)

    agg3 = _sc_agg(ts3.reshape(4 * NP, 16), srcoff4, dst2).reshape(4, NP, 16)
    pre3, st3 = _tc_stats(agg3, dinv, b3r)
    out = _tc_final(pre3, st3, g3r, be3r, batch2, fc1_W, fc1b, fc2p, fc2bp)
    return out[:, :1]


# speedup vs baseline: 14.4594x; 1.0245x over previous
import functools

import jax
import jax.numpy as jnp
from jax import lax
from jax.experimental import pallas as pl
from jax.experimental.pallas import tpu as pltpu
from jax.experimental.pallas import tpu_sc as plsc

N = 50000
NP = 50176
E = 800000
R = 6400
H = 64
NG = 32
EPS = 1e-5
BN = 6272
NBLK = NP // BN
NPT = NP // 16
K = 8

_f32 = jnp.float32
_i32 = jnp.int32


def _sc_mesh():
    return plsc.VectorSubcoreMesh(core_axis_name="c", subcore_axis_name="s")


def _sc_deg(dst2):
    @functools.partial(
        pl.kernel,
        out_type=jax.ShapeDtypeStruct((2 * NP,), _f32),
        mesh=_sc_mesh(),
        compiler_params=pltpu.CompilerParams(use_tc_tiling_on_sc=False),
        scratch_types=[
            pltpu.VMEM((NPT,), _f32),
            pltpu.VMEM((128,), _f32),
            pltpu.VMEM((K, 128), _i32),
            pltpu.VMEM_SHARED((NP,), _f32),
        ],
    )
    def k(dst2_hbm, degp_hbm, zbuf, ones_v, dstv, degp_sh):
        c = lax.axis_index("c")
        s = lax.axis_index("s")
        @pl.loop(0, NPT // 16)
        def _z(i):
            zbuf[pl.ds(i * 16, 16)] = jnp.zeros((16,), _f32)
        for j in range(8):
            ones_v[pl.ds(j * 16, 16)] = jnp.ones((16,), _f32)
        base_n = s * NPT
        pltpu.sync_copy(zbuf, degp_sh.at[pl.ds(base_n, NPT)])
        plsc.subcore_barrier()
        rows_per_tile = R // 32
        base_r = c * (R // 2) + s * rows_per_tile
        @pl.loop(0, rows_per_tile // K)
        def _chunk(kk):
            r0 = base_r + kk * K
            pltpu.sync_copy(dst2_hbm.at[pl.ds(r0, K)], dstv)
            for j in range(K):
                pltpu.sync_copy(ones_v.at[pl.ds(0, 128)],
                                degp_sh.at[dstv.at[j]], add=True)
        plsc.subcore_barrier()
        pltpu.sync_copy(degp_sh.at[pl.ds(base_n, NPT)], zbuf)
        pltpu.sync_copy(zbuf, degp_hbm.at[pl.ds(c * NP + base_n, NPT)])

    return k(dst2).reshape(2, NP)


def _sc_agg(ts_flat, srcoff, dst2, groups, cols, kk_rows, nbuf):
    K_ = kk_rows
    nchunk = (R // 16) // K_
    per_core = groups // 2
    front = 1
    while (nchunk - nbuf + 1 - front) % nbuf:
        front += 1

    @functools.partial(
        pl.kernel,
        out_type=jax.ShapeDtypeStruct((groups * NP, cols), _f32),
        mesh=_sc_mesh(),
        compiler_params=pltpu.CompilerParams(use_tc_tiling_on_sc=False),
        scratch_types=[
            pltpu.VMEM((nbuf * K_ * 128, cols), _f32),
            pltpu.VMEM((nbuf * K_, 128), _i32),
            pltpu.VMEM((nbuf * K_, 128), _i32),
        ] + [pltpu.SemaphoreType.DMA] * (2 * nbuf) + [
            pltpu.VMEM_SHARED((NP, cols), _f32),
        ],
    )
    def k(ts_hbm, src_hbm, dst_hbm, out_hbm, rows_v, srcv, dstv, *sems):
        acc = sems[-1]
        gsem = sems[:nbuf]
        ssem = sems[nbuf:2 * nbuf]
        c = lax.axis_index("c")
        s = lax.axis_index("s")
        base_n = s * NPT
        ch = NPT // 4
        base_r = s * (R // 16)
        for p in range(per_core):
            g = per_core * c + p

            def load_idx(kk, b):
                r0 = base_r + kk * K_
                pltpu.sync_copy(src_hbm.at[g, pl.ds(r0, K_)],
                                srcv.at[pl.ds(b * K_, K_)])
                pltpu.sync_copy(dst_hbm.at[pl.ds(r0, K_)],
                                dstv.at[pl.ds(b * K_, K_)])

            def gathers(b, fire):
                for j in range(K_):
                    cp = pltpu.make_async_copy(
                        ts_hbm.at[srcv.at[b * K_ + j]],
                        rows_v.at[pl.ds((b * K_ + j) * 128, 128)], gsem[b])
                    cp.start() if fire else cp.wait()

            def scatters(b, fire):
                for j in range(K_):
                    src_sl = rows_v.at[pl.ds((b * K_ + j) * 128, 128)]
                    dst_sl = acc.at[dstv.at[b * K_ + j]]
                    if fire:
                        pltpu.async_copy(src_sl, dst_sl, ssem[b], add=True)
                    else:
                        pltpu.make_async_copy(src_sl, dst_sl, ssem[b]).wait()

            def body(kk, b, drain=True, prefetch=True):
                bn = (b + nbuf - 1) % nbuf
                if drain:
                    scatters(bn, False)
                if prefetch:
                    load_idx(kk + nbuf - 1, bn)
                    gathers(bn, True)
                gathers(b, False)
                scatters(b, True)

            for q in range(4):
                pltpu.sync_copy(
                    ts_hbm.at[pl.ds(g * NP + base_n + q * ch, ch)],
                    rows_v.at[pl.ds(0, ch)])
                pltpu.sync_copy(rows_v.at[pl.ds(0, ch)],
                                acc.at[pl.ds(base_n + q * ch, ch)])
            plsc.subcore_barrier()
            for t in range(nbuf - 1):
                load_idx(t, t)
                gathers(t, True)
            body(0, 0, drain=False)
            for kk in range(1, front):
                body(kk, kk % nbuf)
            @pl.loop(0, (nchunk - nbuf + 1 - front) // nbuf)
            def _grp(it):
                kk0 = front + nbuf * it
                for u in range(nbuf):
                    body(kk0 + u, (front + u) % nbuf)
            for kk in range(nchunk - nbuf + 1, nchunk):
                body(kk, kk % nbuf, prefetch=False)
            scatters((nchunk - 1) % nbuf, False)
            plsc.subcore_barrier()
            for q in range(4):
                pltpu.sync_copy(acc.at[pl.ds(base_n + q * ch, ch)],
                                rows_v.at[pl.ds(0, ch)])
                pltpu.sync_copy(
                    rows_v.at[pl.ds(0, ch)],
                    out_hbm.at[pl.ds(g * NP + base_n + q * ch, ch)])

    return k(ts_flat, srcoff, dst2)


GROUPS = 2
COLS = H // GROUPS
SC_K = 2
SC_NBUF = 3


def _splitg(ts):
    return jnp.concatenate(
        [ts[None, :, COLS * g:COLS * (g + 1)] for g in range(GROUPS)], 0)


def _tc_prep(degpT, xp, W1p):
    def body(degp_ref, x_ref, w_ref, dinv_ref, ts_ref):
        deg = degp_ref[:, 0:1] + degp_ref[:, 1:2] + 1.0
        di = 1.0 / jnp.sqrt(deg)
        dinv_ref[...] = di
        t = jnp.dot(x_ref[...], w_ref[...], preferred_element_type=_f32, precision=lax.Precision.HIGHEST)
        ts_ref[...] = _splitg(t * di)

    return pl.pallas_call(
        body,
        grid=(NBLK,),
        in_specs=[
            pl.BlockSpec((BN, 2), lambda i: (i, 0)),
            pl.BlockSpec((BN, 8), lambda i: (i, 0)),
            pl.BlockSpec((8, 64), lambda i: (0, 0)),
        ],
        out_specs=[
            pl.BlockSpec((BN, 1), lambda i: (i, 0)),
            pl.BlockSpec((GROUPS, BN, COLS), lambda i: (0, i, 0)),
        ],
        out_shape=[
            jax.ShapeDtypeStruct((NP, 1), _f32),
            jax.ShapeDtypeStruct((GROUPS, NP, COLS), _f32),
        ],
    )(degpT, xp, W1p)


def _tc_stats(agg, dinv, b):
    def body(*refs):
        ags = refs[:GROUPS]
        dinv_ref, b_ref, pre_ref, st_ref = refs[GROUPS:]
        i = pl.program_id(0)
        a = jnp.concatenate([ar[0] for ar in ags], axis=-1)
        pre = a * dinv_ref[...] + b_ref[...]
        pre_ref[...] = pre
        rows = i * BN + lax.broadcasted_iota(_i32, (BN, 1), 0)
        pm = jnp.where(rows < N, pre, 0.0)
        st = jnp.concatenate([jnp.sum(pm, 0)[None], jnp.sum(pm * pm, 0)[None]],
                             0)
        @pl.when(i == 0)
        def _():
            st_ref[...] = st
        @pl.when(i > 0)
        def _():
            st_ref[...] += st

    gspec = lambda g: pl.BlockSpec((1, BN, COLS), lambda i, g=g: (g, i, 0))
    return pl.pallas_call(
        body,
        grid=(NBLK,),
        in_specs=[gspec(g) for g in range(GROUPS)] + [
            pl.BlockSpec((BN, 1), lambda i: (i, 0)),
            pl.BlockSpec((1, 64), lambda i: (0, 0)),
        ],
        out_specs=[
            pl.BlockSpec((BN, 64), lambda i: (i, 0)),
            pl.BlockSpec((2, 64), lambda i: (0, 0)),
        ],
        out_shape=[
            jax.ShapeDtypeStruct((NP, 64), _f32),
            jax.ShapeDtypeStruct((2, 64), _f32),
        ],
    )(*([agg] * GROUPS), dinv, b)


def _bn_relu(pre_ref, st_ref, g_ref, be_ref, i):
    m = st_ref[0:1, :] * (1.0 / N)
    v = st_ref[1:2, :] * (1.0 / N) - m * m
    scale = g_ref[...] / jnp.sqrt(v + EPS)
    shift = be_ref[...] - m * scale
    h = jnp.maximum(pre_ref[...] * scale + shift, 0.0)
    rows = i * BN + lax.broadcasted_iota(_i32, (BN, 1), 0)
    return jnp.where(rows < N, h, 0.0)


def _tc_apply_mid(pre, stats, dinv, g, be, Wn):
    def body(pre_ref, st_ref, dinv_ref, g_ref, be_ref, w_ref, ts_ref):
        i = pl.program_id(0)
        h = _bn_relu(pre_ref, st_ref, g_ref, be_ref, i)
        t = jnp.dot(h, w_ref[...], preferred_element_type=_f32, precision=lax.Precision.HIGHEST)
        ts_ref[...] = _splitg(t * dinv_ref[...])

    return pl.pallas_call(
        body,
        grid=(NBLK,),
        in_specs=[
            pl.BlockSpec((BN, 64), lambda i: (i, 0)),
            pl.BlockSpec((2, 64), lambda i: (0, 0)),
            pl.BlockSpec((BN, 1), lambda i: (i, 0)),
            pl.BlockSpec((1, 64), lambda i: (0, 0)),
            pl.BlockSpec((1, 64), lambda i: (0, 0)),
            pl.BlockSpec((64, 64), lambda i: (0, 0)),
        ],
        out_specs=[pl.BlockSpec((GROUPS, BN, COLS), lambda i: (0, i, 0))],
        out_shape=[jax.ShapeDtypeStruct((GROUPS, NP, COLS), _f32)],
    )(pre, stats, dinv, g, be, Wn)[0]


def _tc_final(pre, stats, g, be, batch2, fc1_W, fc1_b, fc2p, fc2bp):
    def body(pre_ref, st_ref, g_ref, be_ref, bt_ref, w1_ref, b1_ref, w2_ref,
             b2_ref, pool_ref, cnt_ref, out_ref):
        i = pl.program_id(0)
        h = _bn_relu(pre_ref, st_ref, g_ref, be_ref, i)
        oh = (bt_ref[...] == lax.broadcasted_iota(_i32, (1, NG), 1))
        oh = oh.astype(_f32)
        psum = lax.dot_general(oh, h, (((0,), (0,)), ((), ())),
                               preferred_element_type=_f32, precision=lax.Precision.HIGHEST)
        csum = lax.dot_general(oh, jnp.ones((BN, H), _f32),
                               (((0,), (0,)), ((), ())),
                               preferred_element_type=_f32, precision=lax.Precision.HIGHEST)
        @pl.when(i == 0)
        def _():
            pool_ref[...] = psum
            cnt_ref[...] = csum
        @pl.when(i > 0)
        def _():
            pool_ref[...] += psum
            cnt_ref[...] += csum
        pooled = pool_ref[...] / jnp.maximum(cnt_ref[...], 1.0)
        z = jnp.maximum(
            jnp.dot(pooled, w1_ref[...], preferred_element_type=_f32, precision=lax.Precision.HIGHEST)
            + b1_ref[...], 0.0)
        out_ref[...] = jnp.dot(z, w2_ref[...],
                               preferred_element_type=_f32, precision=lax.Precision.HIGHEST) + b2_ref[...]

    return pl.pallas_call(
        body,
        grid=(NBLK,),
        in_specs=[
            pl.BlockSpec((BN, 64), lambda i: (i, 0)),
            pl.BlockSpec((2, 64), lambda i: (0, 0)),
            pl.BlockSpec((1, 64), lambda i: (0, 0)),
            pl.BlockSpec((1, 64), lambda i: (0, 0)),
            pl.BlockSpec((BN, 1), lambda i: (i, 0)),
            pl.BlockSpec((64, 64), lambda i: (0, 0)),
            pl.BlockSpec((1, 64), lambda i: (0, 0)),
            pl.BlockSpec((64, 128), lambda i: (0, 0)),
            pl.BlockSpec((1, 128), lambda i: (0, 0)),
        ],
        out_specs=[
            pl.BlockSpec((NG, 64), lambda i: (0, 0)),
            pl.BlockSpec((NG, 64), lambda i: (0, 0)),
            pl.BlockSpec((NG, 128), lambda i: (0, 0)),
        ],
        out_shape=[
            jax.ShapeDtypeStruct((NG, 64), _f32),
            jax.ShapeDtypeStruct((NG, 64), _f32),
            jax.ShapeDtypeStruct((NG, 128), _f32),
        ],
    )(pre, stats, g, be, batch2, fc1_W, fc1_b, fc2p, fc2bp)[2]


def kernel(x, edge_index, batch, W1, b1, g1, be1, W2, b2, g2, be2, W3, b3, g3,
           be3, fc1_W, fc1_b, fc2_W, fc2_b):
    src = edge_index[0]
    dst = edge_index[1]
    epad = R * 128 - E
    src2 = jnp.concatenate([src, jnp.full((epad,), N, _i32)]).reshape(R, 128)
    dst2 = jnp.concatenate([dst, jnp.full((epad,), N, _i32)]).reshape(R, 128)
    srcoff = jnp.concatenate([(src2 + g * NP)[None] for g in range(GROUPS)], 0)

    xp = jnp.pad(x, ((0, NP - N), (0, 2)))
    W1p = jnp.pad(W1, ((0, 2), (0, 0)))
    batch2 = jnp.pad(batch, (0, NP - N), constant_values=NG).reshape(NP, 1)
    b1r, b2r, b3r = b1.reshape(1, H), b2.reshape(1, H), b3.reshape(1, H)
    g1r, g2r, g3r = g1.reshape(1, H), g2.reshape(1, H), g3.reshape(1, H)
    be1r, be2r, be3r = be1.reshape(1, H), be2.reshape(1, H), be3.reshape(1, H)
    fc1b = fc1_b.reshape(1, H)
    fc2p = jnp.pad(fc2_W, ((0, 0), (0, 127)))
    fc2bp = jnp.pad(fc2_b.reshape(1, 1), ((0, 0), (0, 127)))

    degp = _sc_deg(dst2)
    dinv, ts = _tc_prep(degp.T, xp, W1p)

    def agg_call(t):
        a = _sc_agg(t.reshape(GROUPS * NP, COLS), srcoff, dst2,
                    GROUPS, COLS, SC_K, SC_NBUF)
        return a.reshape(GROUPS, NP, COLS)

    pre1, st1 = _tc_stats(agg_call(ts), dinv, b1r)
    ts2 = _tc_apply_mid(pre1, st1, dinv, g1r, be1r, W2)

    pre2, st2 = _tc_stats(agg_call(ts2), dinv, b2r)
    ts3 = _tc_apply_mid(pre2, st2, dinv, g2r, be2r, W3)

    pre3, st3 = _tc_stats(agg_call(ts3), dinv, b3r)
    out = _tc_final(pre3, st3, g3r, be3r, batch2, fc1_W, fc1b, fc2p, fc2bp)
    return out[:, :1]
